# Initial kernel scaffold; baseline (speedup 1.0000x reference)
import functools

import jax
import jax.numpy as jnp
from jax import lax
from jax.experimental import pallas as pl
from jax.experimental.pallas import tpu as pltpu
from jax.experimental.pallas import tpu_sc as plsc

B = 16384
D = 128
K = 30
EPS_NORM = 1e-12

NW = 32
ROWS_PW = B // NW
IDX_PW = ROWS_PW * K
ROWS_SUPER = 64
SUPER = ROWS_PW // ROWS_SUPER
CHUNK_ROWS = 4
CHUNK_IDX = CHUNK_ROWS * K
CHUNKS_PER_SUPER = ROWS_SUPER // CHUNK_ROWS
LANES = 16
NCHUNK16 = D // LANES

BB = 1024


def _normalize_body(zi_ref, zj_ref, out_ref):
    for s, ref in ((0, zi_ref), (1, zj_ref)):
        x = ref[...]
        n = jnp.sqrt(jnp.sum(x * x, axis=1, keepdims=True))
        out_ref[s] = x / jnp.maximum(n, EPS_NORM)


_normalize = pl.pallas_call(
    _normalize_body,
    grid=(B // BB,),
    in_specs=[
        pl.BlockSpec((BB, D), lambda i: (i, 0)),
        pl.BlockSpec((BB, D), lambda i: (i, 0)),
    ],
    out_specs=pl.BlockSpec((2, BB, D), lambda i: (0, i, 0)),
    out_shape=jax.ShapeDtypeStruct((2, B, D), jnp.float32),
)


@functools.partial(
    pl.kernel,
    out_type=jax.ShapeDtypeStruct((B * K,), jnp.float32),
    mesh=plsc.VectorSubcoreMesh(core_axis_name="c", subcore_axis_name="s"),
    scratch_types=[
        pltpu.VMEM((IDX_PW,), jnp.int32),
        pltpu.VMEM((ROWS_SUPER, D), jnp.float32),
        pltpu.VMEM((CHUNK_IDX, D), jnp.float32),
        pltpu.VMEM((IDX_PW,), jnp.float32),
        pltpu.SemaphoreType.DMA,
    ],
)
def _sc_negdot(pool_hbm, idx_hbm, out_hbm, idx_v, zi_v, rows_v, out_v, sem):
    wid = lax.axis_index("s") * 2 + lax.axis_index("c")
    idx_base = wid * IDX_PW
    row_base = wid * ROWS_PW
    pltpu.sync_copy(idx_hbm.at[pl.ds(idx_base, IDX_PW)], idx_v)

    def super_body(s, carry):
        pltpu.sync_copy(
            pool_hbm.at[pl.ds(row_base + s * ROWS_SUPER, ROWS_SUPER)], zi_v
        )

        def chunk_body(cc, carry2):
            off = s * (ROWS_SUPER * K) + cc * CHUNK_IDX
            pltpu.async_copy(
                pool_hbm.at[idx_v.at[pl.ds(off, CHUNK_IDX)]], rows_v, sem
            ).wait()
            for j in range(CHUNK_ROWS):
                r = cc * CHUNK_ROWS + j
                zi_regs = [
                    zi_v[r, pl.ds(LANES * t, LANES)] for t in range(NCHUNK16)
                ]
                for k in range(K):
                    g = j * K + k
                    acc = rows_v[g, pl.ds(0, LANES)] * zi_regs[0]
                    for t in range(1, NCHUNK16):
                        acc = acc + rows_v[g, pl.ds(LANES * t, LANES)] * zi_regs[t]
                    out_v[off + g] = jnp.sum(acc)
            return carry2

        return lax.fori_loop(0, CHUNKS_PER_SUPER, chunk_body, carry)

    lax.fori_loop(0, SUPER, super_body, 0)
    pltpu.sync_copy(out_v, out_hbm.at[pl.ds(idx_base, IDX_PW)])


def _loss_body(temp_ref, pool_ref, neg_ref, out_ref):
    i = pl.program_id(0)
    zi = pool_ref[0]
    zj = pool_ref[1]
    inv_t = 1.0 / temp_ref[0]
    pos = jnp.sum(zi * zj, axis=1, keepdims=True)
    logits = jnp.concatenate([pos, neg_ref[...]], axis=1) * inv_t
    m = jnp.max(logits, axis=1, keepdims=True)
    lse = m[:, 0] + jnp.log(jnp.sum(jnp.exp(logits - m), axis=1))
    contrib = jnp.sum(lse - pos[:, 0] * inv_t) * (1.0 / B)

    @pl.when(i == 0)
    def _():
        out_ref[0, 0] = 0.0

    out_ref[0, 0] += contrib


_loss = pl.pallas_call(
    _loss_body,
    grid=(B // BB,),
    in_specs=[
        pl.BlockSpec(memory_space=pltpu.SMEM),
        pl.BlockSpec((2, BB, D), lambda i: (0, i, 0)),
        pl.BlockSpec((BB, K), lambda i: (i, 0)),
    ],
    out_specs=pl.BlockSpec((1, 1), lambda i: (0, 0)),
    out_shape=jax.ShapeDtypeStruct((1, 1), jnp.float32),
)


def kernel(z_i, z_j, temperature, neg_indices):
    pool3 = _normalize(z_i, z_j)
    pool = pool3.reshape(2 * B, D)
    neg_flat = _sc_negdot(pool, neg_indices.reshape(B * K))
    neg = neg_flat.reshape(B, K)
    out = _loss(temperature.reshape(1), pool3, neg)
    return out[0, 0]


# speedup vs baseline: 5.6578x; 5.6578x over previous
import functools

import jax
import jax.numpy as jnp
from jax import lax
from jax.experimental import pallas as pl
from jax.experimental.pallas import tpu as pltpu
from jax.experimental.pallas import tpu_sc as plsc

B = 16384
D = 128
K = 30
KP = 32
EPS_NORM = 1e-12

NW = 32
ROWS_PW = B // NW
IDX_PW = ROWS_PW * KP
ROWS_SUPER = 64
SUPER = ROWS_PW // ROWS_SUPER
CHUNK_ROWS = 4
CHUNK_IDX = CHUNK_ROWS * KP
CHUNKS_PER_SUPER = ROWS_SUPER // CHUNK_ROWS
LANES = 16
NCHUNK16 = D // LANES

BB = 1024


def _normalize_body(zi_ref, zj_ref, out_ref):
    for s, ref in ((0, zi_ref), (1, zj_ref)):
        x = ref[...]
        n = jnp.sqrt(jnp.sum(x * x, axis=1, keepdims=True))
        out_ref[s] = x / jnp.maximum(n, EPS_NORM)


_normalize = pl.pallas_call(
    _normalize_body,
    grid=(B // BB,),
    in_specs=[
        pl.BlockSpec((BB, D), lambda i: (i, 0)),
        pl.BlockSpec((BB, D), lambda i: (i, 0)),
    ],
    out_specs=pl.BlockSpec((2, BB, D), lambda i: (0, i, 0)),
    out_shape=jax.ShapeDtypeStruct((2, B, D), jnp.float32),
)


_GATHER_1D = lax.GatherDimensionNumbers(
    offset_dims=(), collapsed_slice_dims=(0,), start_index_map=(0,)
)


def _lane_perm(x, perm_2d):
    return lax.gather(
        x,
        perm_2d,
        dimension_numbers=_GATHER_1D,
        slice_sizes=(1,),
        mode=lax.GatherScatterMode.PROMISE_IN_BOUNDS,
    )


def _lane_sum_16(accs, lane_iota):
    vecs = list(accs)
    for lev in range(4):
        h = 1 << lev
        perm = (lane_iota ^ h)[:, None]
        upper = (lane_iota & h) != 0
        nxt = []
        for p in range(0, len(vecs), 2):
            ta = vecs[p] + _lane_perm(vecs[p], perm)
            tb = vecs[p + 1] + _lane_perm(vecs[p + 1], perm)
            nxt.append(jnp.where(upper, tb, ta))
        vecs = nxt
    return vecs[0]


def _sc_negdot_body(pool_hbm, idx_hbm, out_hbm, idx_v, zi_v, rows_v, out_v, sem):
    wid = lax.axis_index("s") * 2 + lax.axis_index("c")
    idx_base = wid * IDX_PW
    row_base = wid * ROWS_PW
    lane_iota = lax.iota(jnp.int32, LANES)
    pltpu.sync_copy(idx_hbm.at[pl.ds(idx_base, IDX_PW)], idx_v)

    def super_body(s, carry):
        pltpu.sync_copy(
            pool_hbm.at[pl.ds(row_base + s * ROWS_SUPER, ROWS_SUPER)], zi_v
        )

        def chunk_body(cc, carry2):
            off = s * (ROWS_SUPER * KP) + cc * CHUNK_IDX
            pltpu.async_copy(
                pool_hbm.at[idx_v.at[pl.ds(off, CHUNK_IDX)]], rows_v, sem
            ).wait()
            for j in range(CHUNK_ROWS):
                r = cc * CHUNK_ROWS + j
                zi_regs = [
                    zi_v[r, pl.ds(LANES * t, LANES)] for t in range(NCHUNK16)
                ]
                for grp in range(KP // LANES):
                    accs = []
                    for l in range(LANES):
                        g = j * KP + grp * LANES + l
                        acc = rows_v[g, pl.ds(0, LANES)] * zi_regs[0]
                        for t in range(1, NCHUNK16):
                            acc = acc + rows_v[g, pl.ds(LANES * t, LANES)] * zi_regs[t]
                        accs.append(acc)
                    dots = _lane_sum_16(accs, lane_iota)
                    out_v[pl.ds(off + j * KP + grp * LANES, LANES)] = dots
            return carry2

        return lax.fori_loop(0, CHUNKS_PER_SUPER, chunk_body, carry)

    lax.fori_loop(0, SUPER, super_body, 0)
    pltpu.sync_copy(out_v, out_hbm.at[pl.ds(idx_base, IDX_PW)])


@functools.cache
def _get_sc_negdot():
    return pl.kernel(
        _sc_negdot_body,
        out_type=jax.ShapeDtypeStruct((B * KP,), jnp.float32),
        mesh=plsc.VectorSubcoreMesh(core_axis_name="c", subcore_axis_name="s"),
        scratch_types=[
            pltpu.VMEM((IDX_PW,), jnp.int32),
            pltpu.VMEM((ROWS_SUPER, D), jnp.float32),
            pltpu.VMEM((CHUNK_IDX, D), jnp.float32),
            pltpu.VMEM((IDX_PW,), jnp.float32),
            pltpu.SemaphoreType.DMA,
        ],
    )


def _loss_body(temp_ref, pool_ref, neg_ref, out_ref):
    i = pl.program_id(0)
    zi = pool_ref[0]
    zj = pool_ref[1]
    inv_t = 1.0 / temp_ref[0]
    pos = jnp.sum(zi * zj, axis=1, keepdims=True)
    neg = neg_ref[...][:, :K]
    logits = jnp.concatenate([pos, neg], axis=1) * inv_t
    m = jnp.max(logits, axis=1, keepdims=True)
    lse = m[:, 0] + jnp.log(jnp.sum(jnp.exp(logits - m), axis=1))
    contrib = jnp.sum(lse - pos[:, 0] * inv_t) * (1.0 / B)

    @pl.when(i == 0)
    def _():
        out_ref[...] = jnp.zeros_like(out_ref)

    out_ref[...] += jnp.full((1, 1), contrib, dtype=jnp.float32)


_loss = pl.pallas_call(
    _loss_body,
    grid=(B // BB,),
    in_specs=[
        pl.BlockSpec(memory_space=pltpu.SMEM),
        pl.BlockSpec((2, BB, D), lambda i: (0, i, 0)),
        pl.BlockSpec((BB, KP), lambda i: (i, 0)),
    ],
    out_specs=pl.BlockSpec((1, 1), lambda i: (0, 0)),
    out_shape=jax.ShapeDtypeStruct((1, 1), jnp.float32),
)


def kernel(z_i, z_j, temperature, neg_indices):
    pool3 = _normalize(z_i, z_j)
    pool = pool3.reshape(2 * B, D)
    idx32 = jnp.concatenate([neg_indices, neg_indices[:, :2]], axis=1)
    neg_flat = _get_sc_negdot()(pool, idx32.reshape(B * KP))
    neg = neg_flat.reshape(B, KP)
    out = _loss(temperature.reshape(1), pool3, neg)
    return out[0, 0]


# speedup vs baseline: 8.9946x; 1.5898x over previous
import functools

import jax
import jax.numpy as jnp
from jax import lax
from jax.experimental import pallas as pl
from jax.experimental.pallas import tpu as pltpu
from jax.experimental.pallas import tpu_sc as plsc

B = 16384
D = 128
K = 30
KP = 32
EPS_NORM = 1e-12

NW = 32
ROWS_PW = B // NW
IDX_PW = ROWS_PW * KP
ROWS_SUPER = 64
SUPER = ROWS_PW // ROWS_SUPER
CHUNK_ROWS = 4
CHUNK_IDX = CHUNK_ROWS * KP
CHUNKS_PER_SUPER = ROWS_SUPER // CHUNK_ROWS
LANES = 16
NCHUNK16 = D // LANES

BB = 1024


def _normalize_body(zi_ref, zj_ref, out_ref):
    for s, ref in ((0, zi_ref), (1, zj_ref)):
        x = ref[...]
        n = jnp.sqrt(jnp.sum(x * x, axis=1, keepdims=True))
        out_ref[s] = x / jnp.maximum(n, EPS_NORM)


_normalize = pl.pallas_call(
    _normalize_body,
    grid=(B // BB,),
    in_specs=[
        pl.BlockSpec((BB, D), lambda i: (i, 0)),
        pl.BlockSpec((BB, D), lambda i: (i, 0)),
    ],
    out_specs=pl.BlockSpec((2, BB, D), lambda i: (0, i, 0)),
    out_shape=jax.ShapeDtypeStruct((2, B, D), jnp.float32),
)


_GATHER_1D = lax.GatherDimensionNumbers(
    offset_dims=(), collapsed_slice_dims=(0,), start_index_map=(0,)
)


def _lane_perm(x, perm_2d):
    return lax.gather(
        x,
        perm_2d,
        dimension_numbers=_GATHER_1D,
        slice_sizes=(1,),
        mode=lax.GatherScatterMode.PROMISE_IN_BOUNDS,
    )


def _lane_sum_16(accs, lane_iota):
    vecs = list(accs)
    for lev in range(4):
        h = 1 << lev
        perm = (lane_iota ^ h)[:, None]
        upper = (lane_iota & h) != 0
        nxt = []
        for p in range(0, len(vecs), 2):
            ta = vecs[p] + _lane_perm(vecs[p], perm)
            tb = vecs[p + 1] + _lane_perm(vecs[p + 1], perm)
            nxt.append(jnp.where(upper, tb, ta))
        vecs = nxt
    return vecs[0]


NCHUNKS = IDX_PW // CHUNK_IDX


def _sc_negdot_body(
    pool_hbm, idx_hbm, out_hbm, idx_v, zi_v, rows_v, out_v, sem0, sem1
):
    wid = lax.axis_index("s") * 2 + lax.axis_index("c")
    idx_base = wid * IDX_PW
    row_base = wid * ROWS_PW
    lane_iota = lax.iota(jnp.int32, LANES)
    pltpu.sync_copy(idx_hbm.at[pl.ds(idx_base, IDX_PW)], idx_v)

    def fire(c):
        src = pool_hbm.at[idx_v.at[pl.ds(c * CHUNK_IDX, CHUNK_IDX)]]

        @pl.when((c & 1) == 0)
        def _():
            pltpu.async_copy(src, rows_v.at[0], sem0)

        @pl.when((c & 1) == 1)
        def _():
            pltpu.async_copy(src, rows_v.at[1], sem1)

    fire(0)

    def chunk_body(cc, carry):
        @pl.when(lax.rem(cc, CHUNKS_PER_SUPER) == 0)
        def _():
            pltpu.sync_copy(
                pool_hbm.at[
                    pl.ds(row_base + (cc // CHUNKS_PER_SUPER) * ROWS_SUPER,
                          ROWS_SUPER)
                ],
                zi_v,
            )

        @pl.when(cc + 1 < NCHUNKS)
        def _():
            fire(cc + 1)

        dummy = pool_hbm.at[pl.ds(0, CHUNK_IDX)]

        @pl.when((cc & 1) == 0)
        def _():
            pltpu.make_async_copy(dummy, rows_v.at[0], sem0).wait()

        @pl.when((cc & 1) == 1)
        def _():
            pltpu.make_async_copy(dummy, rows_v.at[1], sem1).wait()

        par = cc & 1
        off = cc * CHUNK_IDX
        rloc = lax.rem(cc, CHUNKS_PER_SUPER) * CHUNK_ROWS
        for j in range(CHUNK_ROWS):
            zi_regs = [
                zi_v[rloc + j, pl.ds(LANES * t, LANES)] for t in range(NCHUNK16)
            ]
            for grp in range(KP // LANES):
                accs = []
                for l in range(LANES):
                    g = j * KP + grp * LANES + l
                    acc = rows_v[par, g, pl.ds(0, LANES)] * zi_regs[0]
                    for t in range(1, NCHUNK16):
                        acc = acc + rows_v[par, g, pl.ds(LANES * t, LANES)] * zi_regs[t]
                    accs.append(acc)
                dots = _lane_sum_16(accs, lane_iota)
                out_v[pl.ds(off + j * KP + grp * LANES, LANES)] = dots
        return carry

    lax.fori_loop(0, NCHUNKS, chunk_body, 0)
    pltpu.sync_copy(out_v, out_hbm.at[pl.ds(idx_base, IDX_PW)])


@functools.cache
def _get_sc_negdot():
    return pl.kernel(
        _sc_negdot_body,
        out_type=jax.ShapeDtypeStruct((B * KP,), jnp.float32),
        mesh=plsc.VectorSubcoreMesh(core_axis_name="c", subcore_axis_name="s"),
        scratch_types=[
            pltpu.VMEM((IDX_PW,), jnp.int32),
            pltpu.VMEM((ROWS_SUPER, D), jnp.float32),
            pltpu.VMEM((2, CHUNK_IDX, D), jnp.float32),
            pltpu.VMEM((IDX_PW,), jnp.float32),
            pltpu.SemaphoreType.DMA,
            pltpu.SemaphoreType.DMA,
        ],
    )


def _loss_body(temp_ref, pool_ref, neg_ref, out_ref):
    i = pl.program_id(0)
    zi = pool_ref[0]
    zj = pool_ref[1]
    inv_t = 1.0 / temp_ref[0]
    pos = jnp.sum(zi * zj, axis=1, keepdims=True)
    neg = neg_ref[...][:, :K]
    logits = jnp.concatenate([pos, neg], axis=1) * inv_t
    m = jnp.max(logits, axis=1, keepdims=True)
    lse = m[:, 0] + jnp.log(jnp.sum(jnp.exp(logits - m), axis=1))
    contrib = jnp.sum(lse - pos[:, 0] * inv_t) * (1.0 / B)

    @pl.when(i == 0)
    def _():
        out_ref[...] = jnp.zeros_like(out_ref)

    out_ref[...] += jnp.full((1, 1), contrib, dtype=jnp.float32)


_loss = pl.pallas_call(
    _loss_body,
    grid=(B // BB,),
    in_specs=[
        pl.BlockSpec(memory_space=pltpu.SMEM),
        pl.BlockSpec((2, BB, D), lambda i: (0, i, 0)),
        pl.BlockSpec((BB, KP), lambda i: (i, 0)),
    ],
    out_specs=pl.BlockSpec((1, 1), lambda i: (0, 0)),
    out_shape=jax.ShapeDtypeStruct((1, 1), jnp.float32),
)


def kernel(z_i, z_j, temperature, neg_indices):
    pool3 = _normalize(z_i, z_j)
    pool = pool3.reshape(2 * B, D)
    idx32 = jnp.concatenate([neg_indices, neg_indices[:, :2]], axis=1)
    neg_flat = _get_sc_negdot()(pool, idx32.reshape(B * KP))
    neg = neg_flat.reshape(B, KP)
    out = _loss(temperature.reshape(1), pool3, neg)
    return out[0, 0]
